# baseline (speedup 1.0000x reference)
import dataclasses
import functools

import jax
import jax.numpy as jnp
import numpy as np
from jax import lax
from jax.experimental import pallas as pl
from jax.experimental.pallas import tpu as pltpu
from jax.experimental.pallas import tpu_sc as plsc

_H = 64
_C = 128
_NTILES = 32
_LANES = 16


def _proj1_body(x_ref, emb_ref, wq, wk, wv, ws, we1, we2, bias_ref,
                qx_ref, kv_ref, skip_ref, e1_ref, e2_ref):
    xb = x_ref[...]
    f32 = jnp.float32
    e1 = jnp.dot(emb_ref[...], we1[...], preferred_element_type=f32)
    q = jnp.dot(xb, wq[...], preferred_element_type=f32) + bias_ref[0:1, :]
    qx_ref[:, :_H] = q
    qx_ref[:, _H:] = jnp.dot(q, e1.T, preferred_element_type=f32,
                             precision=jax.lax.Precision.HIGHEST)
    kv_ref[:, :_H] = jnp.dot(xb, wk[...], preferred_element_type=f32) + bias_ref[1:2, :]
    kv_ref[:, _H:] = jnp.dot(xb, wv[...], preferred_element_type=f32) + bias_ref[2:3, :]
    skip_ref[...] = jnp.dot(xb, ws[...], preferred_element_type=f32) + bias_ref[3:4, :]

    @pl.when(pl.program_id(0) == 0)
    def _():
        e1_ref[...] = e1
        e2_ref[...] = jnp.dot(emb_ref[...], we2[...], preferred_element_type=f32)


def _combine(acc_ref, skip_ref, e_ref):
    acc = acc_ref[0] + acc_ref[1]
    s = acc[:, _H:]
    den = jnp.sum(s, axis=-1, keepdims=True)
    agg = (acc[:, :_H]
           + jnp.dot(s, e_ref[...], preferred_element_type=jnp.float32,
                     precision=jax.lax.Precision.HIGHEST))
    return jnp.maximum(agg / (den + 1e-16) + skip_ref[...], 0.0)


def _mid_body(acc_ref, skip_ref, e1_ref, e2_ref, wq, wk, wv, ws,
              bias_ref, qx_ref, kv_ref, skip2_ref):
    f32 = jnp.float32
    h = _combine(acc_ref, skip_ref, e1_ref)
    q = jnp.dot(h, wq[...], preferred_element_type=f32) + bias_ref[0:1, :]
    qx_ref[:, :_H] = q
    qx_ref[:, _H:] = jnp.dot(q, e2_ref[...].T, preferred_element_type=f32,
                             precision=jax.lax.Precision.HIGHEST)
    kv_ref[:, :_H] = jnp.dot(h, wk[...], preferred_element_type=f32) + bias_ref[1:2, :]
    kv_ref[:, _H:] = jnp.dot(h, wv[...], preferred_element_type=f32) + bias_ref[2:3, :]
    skip2_ref[...] = jnp.dot(h, ws[...], preferred_element_type=f32) + bias_ref[3:4, :]


def _final_body(acc_ref, skip_ref, e2_ref, wo, bias_ref, out_ref):
    h = _combine(acc_ref, skip_ref, e2_ref)
    out_ref[...] = (jnp.dot(h, wo[...], preferred_element_type=jnp.float32)
                    + bias_ref[0:1, :2])


def _full_spec(shape):
    return pl.BlockSpec(shape, lambda i: tuple(0 for _ in shape))


def _proj1(x, emb, wq, wk, wv, ws, we1, we2, bias):
    n, d = x.shape
    blk = 1024
    grid = (n // blk,)
    row = lambda i: (i, 0)
    return pl.pallas_call(
        _proj1_body,
        grid=grid,
        in_specs=[
            pl.BlockSpec((blk, d), row),
            _full_spec(emb.shape),
            _full_spec(wq.shape), _full_spec(wk.shape),
            _full_spec(wv.shape), _full_spec(ws.shape),
            _full_spec(we1.shape), _full_spec(we2.shape),
            _full_spec(bias.shape),
        ],
        out_specs=[
            pl.BlockSpec((blk, _H + _LANES), row),
            pl.BlockSpec((blk, 2 * _H), row),
            pl.BlockSpec((blk, _H), row),
            _full_spec((16, _H)),
            _full_spec((16, _H)),
        ],
        out_shape=[
            jax.ShapeDtypeStruct((n, _H + _LANES), jnp.float32),
            jax.ShapeDtypeStruct((n, 2 * _H), jnp.float32),
            jax.ShapeDtypeStruct((n, _H), jnp.float32),
            jax.ShapeDtypeStruct((16, _H), jnp.float32),
            jax.ShapeDtypeStruct((16, _H), jnp.float32),
        ],
    )(x, emb, wq, wk, wv, ws, we1, we2, bias)


def _mid(accp, skip, e1, e2, wq, wk, wv, ws, bias):
    n = skip.shape[0]
    blk = 1024
    grid = (n // blk,)
    row = lambda i: (i, 0)
    row3 = lambda i: (0, i, 0)
    return pl.pallas_call(
        _mid_body,
        grid=grid,
        in_specs=[
            pl.BlockSpec((2, blk, _H + _LANES), row3),
            pl.BlockSpec((blk, _H), row),
            _full_spec(e1.shape), _full_spec(e2.shape),
            _full_spec(wq.shape), _full_spec(wk.shape),
            _full_spec(wv.shape), _full_spec(ws.shape),
            _full_spec(bias.shape),
        ],
        out_specs=[
            pl.BlockSpec((blk, _H + _LANES), row),
            pl.BlockSpec((blk, 2 * _H), row),
            pl.BlockSpec((blk, _H), row),
        ],
        out_shape=[
            jax.ShapeDtypeStruct((n, _H + _LANES), jnp.float32),
            jax.ShapeDtypeStruct((n, 2 * _H), jnp.float32),
            jax.ShapeDtypeStruct((n, _H), jnp.float32),
        ],
    )(accp, skip, e1, e2, wq, wk, wv, ws, bias)


def _final(accp, skip, e2, wo, bias):
    n = skip.shape[0]
    blk = 1024
    grid = (n // blk,)
    row = lambda i: (i, 0)
    row3 = lambda i: (0, i, 0)
    return pl.pallas_call(
        _final_body,
        grid=grid,
        in_specs=[
            pl.BlockSpec((2, blk, _H + _LANES), row3),
            pl.BlockSpec((blk, _H), row),
            _full_spec(e2.shape),
            _full_spec(wo.shape),
            _full_spec(bias.shape),
        ],
        out_specs=pl.BlockSpec((blk, 2), row),
        out_shape=jax.ShapeDtypeStruct((n, 2), jnp.float32),
    )(accp, skip, e2, wo, bias)


def _edge_phase(qT, kvT, packed_idx):
    n_nodes = qT.shape[0]
    nchunk = packed_idx.shape[0]
    chunks_per_tile = nchunk // _NTILES
    rows_per_tile = n_nodes // 16
    mesh = plsc.VectorSubcoreMesh(core_axis_name="c", subcore_axis_name="s")
    inv_sqrt_h = np.float32(1.0 / np.sqrt(_H))
    cp = pltpu.CompilerParams()
    for fld, val in (("needs_layout_passes", False),
                     ("use_tc_tiling_on_sc", False)):
        if fld in pltpu.CompilerParams.__dataclass_fields__:
            cp = dataclasses.replace(cp, **{fld: val})

    @functools.partial(
        pl.kernel,
        mesh=mesh,
        compiler_params=cp,
        out_type=jax.ShapeDtypeStruct((2, n_nodes, _H + _LANES), jnp.float32),
        scratch_types=[
            pltpu.VMEM((3, _C), jnp.int32), pltpu.VMEM((3, _C), jnp.int32),
            pltpu.VMEM((_C,), jnp.int32), pltpu.VMEM((_C,), jnp.int32),
            pltpu.VMEM((_C, _H + _LANES), jnp.float32),
            pltpu.VMEM((_C, _H + _LANES), jnp.float32),
            pltpu.VMEM((_C, 2 * _H), jnp.float32),
            pltpu.VMEM((_C, 2 * _H), jnp.float32),
            pltpu.VMEM((_C, _H + _LANES), jnp.float32),
            pltpu.VMEM((_C, _H + _LANES), jnp.float32),
            pltpu.VMEM((_LANES, _LANES), jnp.float32),
            pltpu.VMEM_SHARED((n_nodes, _H + _LANES), jnp.float32),
            pltpu.SemaphoreType.DMA, pltpu.SemaphoreType.DMA,
            pltpu.SemaphoreType.DMA, pltpu.SemaphoreType.DMA,
            pltpu.SemaphoreType.DMA, pltpu.SemaphoreType.DMA,
        ],
    )
    def k(q_hbm, kv_hbm, p_hbm, acc_out,
          pidx0, pidx1, scat0, scat1,
          qbuf0, qbuf1, kvbuf0, kvbuf1, obuf0, obuf1, dots,
          acc_s,
          semi0, semi1, semg0, semg1, sems0, sems1):
        cid = lax.axis_index("c")
        sid = lax.axis_index("s")
        wid = sid * 2 + cid

        pidx = (pidx0, pidx1)
        scat = (scat0, scat1)
        qbuf = (qbuf0, qbuf1)
        kvbuf = (kvbuf0, kvbuf1)
        obuf = (obuf0, obuf1)
        semi = (semi0, semi1)
        semg = (semg0, semg1)
        sems = (sems0, sems1)

        zero16 = jnp.zeros((_LANES,), jnp.float32)

        @pl.loop(0, _C)
        def _(r):
            @pl.loop(0, _H + _LANES, step=_LANES)
            def _(j):
                obuf0[r, pl.ds(j, _LANES)] = zero16

        @pl.loop(0, rows_per_tile // _C)
        def _(t):
            r0 = sid * rows_per_tile + t * _C
            pltpu.sync_copy(obuf0.at[pl.ds(0, _C)], acc_s.at[pl.ds(r0, _C)])

        plsc.subcore_barrier()

        lanes = lax.iota(jnp.int32, _LANES)

        def idx_copies(t, b):
            c = wid + t * _NTILES
            return (
                pltpu.make_async_copy(p_hbm.at[c], pidx[b], semi[b]),
            )

        def gather_copies(b):
            return (
                pltpu.make_async_copy(q_hbm.at[pidx[b].at[1]], qbuf[b], semg[b]),
                pltpu.make_async_copy(kv_hbm.at[pidx[b].at[0]], kvbuf[b], semg[b]),
            )

        def scatter_copies(b):
            return (
                pltpu.make_async_copy(obuf[b], acc_s.at[scat[b]], sems[b]),
            )

        def start(copies, add=False):
            for c in copies:
                c.start(add=add)

        def wait(copies):
            for c in copies:
                c.wait()

        def compute(b):
            @pl.loop(0, _C // _LANES)
            def _(g):
                r0 = g * _LANES
                rows = lanes + r0
                attrv = pidx[b][2, pl.ds(r0, _LANES)]

                for e in range(_LANES):
                    r = r0 + e
                    acc = (qbuf[b][r, pl.ds(0, _LANES)]
                           * kvbuf[b][r, pl.ds(0, _LANES)])
                    for s in range(1, _H // _LANES):
                        acc = acc + (qbuf[b][r, pl.ds(s * _LANES, _LANES)]
                                     * kvbuf[b][r, pl.ds(s * _LANES, _LANES)])
                    dots[e, pl.ds(0, _LANES)] = acc

                alpha = jnp.zeros((_LANES,), jnp.float32)
                for j in range(_LANES):
                    jv = jnp.zeros((_LANES,), jnp.int32) + j
                    alpha = alpha + plsc.load_gather(dots, [lanes, jv])

                qe = plsc.load_gather(qbuf[b], [rows, attrv + _H])
                ex = jnp.exp((alpha + qe) * inv_sqrt_h)

                for e in range(_LANES):
                    r = r0 + e
                    exr = jnp.full((_LANES,), ex[e])
                    obuf[b][r, pl.ds(_H, _LANES)] = jnp.where(
                        lanes == attrv[e], exr, 0.0)
                    for s in range(_H // _LANES):
                        obuf[b][r, pl.ds(s * _LANES, _LANES)] = (
                            exr * kvbuf[b][r, pl.ds(_H + s * _LANES, _LANES)])

        def stage(t, b):
            wait(gather_copies(b))
            @pl.when(t + 1 < chunks_per_tile)
            def _():
                wait(idx_copies(t + 1, 1 - b))
                start(gather_copies(1 - b))
            @pl.when(t >= 2)
            def _():
                wait(scatter_copies(b))
            compute(b)

            @pl.loop(0, _C, step=_LANES)
            def _(r):
                scat[b][pl.ds(r, _LANES)] = pidx[b][1, pl.ds(r, _LANES)]

            start(scatter_copies(b), add=True)

            @pl.when(t + 2 < chunks_per_tile)
            def _():
                start(idx_copies(t + 2, b))

        start(idx_copies(0, 0))
        start(idx_copies(1, 1))
        wait(idx_copies(0, 0))
        start(gather_copies(0))

        @pl.loop(0, chunks_per_tile, step=2)
        def _(i):
            stage(i, 0)
            stage(i + 1, 1)

        wait(scatter_copies(0))
        wait(scatter_copies(1))

        plsc.subcore_barrier()
        r0 = sid * rows_per_tile
        pltpu.sync_copy(acc_s.at[pl.ds(r0, rows_per_tile)],
                        acc_out.at[cid, pl.ds(r0, rows_per_tile)])

    return k(qT, kvT, packed_idx)


def kernel(x, edge_index, edge_attr, emb, W1q, b1q, W1k, b1k, W1v, b1v, W1e,
           W1s, b1s, W2q, b2q, W2k, b2k, W2v, b2v, W2e, W2s, b2s, Wo, bo):
    src = edge_index[0]
    dst = edge_index[1]
    attr = edge_attr.astype(jnp.int32)
    n = x.shape[0]
    npad = -(-n // (16 * _C)) * (16 * _C)
    x = jnp.pad(x, ((0, npad - n), (0, 0)))

    e = src.shape[0]
    epad = -(-e // (2 * _NTILES * _C)) * (2 * _NTILES * _C)
    src = jnp.pad(src, (0, epad - e))
    dst = jnp.pad(dst, (0, epad - e), constant_values=npad - 1)
    attr = jnp.pad(attr, (0, epad - e))
    nchunk = epad // _C
    packed_idx = jnp.stack([src.reshape(nchunk, _C), dst.reshape(nchunk, _C),
                            attr.reshape(nchunk, _C)], axis=1)

    bias1 = jnp.zeros((8, _H), jnp.float32)
    bias1 = bias1.at[0].set(b1q).at[1].set(b1k).at[2].set(b1v).at[3].set(b1s)
    bias2 = jnp.zeros((8, _H), jnp.float32)
    bias2 = bias2.at[0].set(b2q).at[1].set(b2k).at[2].set(b2v).at[3].set(b2s)
    biaso = jnp.zeros((8, _H), jnp.float32)
    biaso = biaso.at[0, :2].set(bo)

    qT1, kvT1, skip1, e1, e2 = _proj1(x, emb, W1q, W1k, W1v, W1s, W1e, W2e,
                                      bias1)
    accp1 = _edge_phase(qT1, kvT1, packed_idx)
    qT2, kvT2, skip2 = _mid(accp1, skip1, e1, e2, W2q, W2k, W2v, W2s, bias2)
    accp2 = _edge_phase(qT2, kvT2, packed_idx)
    return _final(accp2, skip2, e2, Wo, biaso)[:n]
